# baseline (speedup 1.0000x reference)
import functools

import jax
import jax.numpy as jnp
from jax import lax
from jax.experimental import pallas as pl
from jax.experimental.pallas import tpu as pltpu
from jax.experimental.pallas import tpu_sc as plsc

B = 16384
D = 64
T = 1000

NC = 2
NS = 16
L = 16
NW = NC * NS
RW = B // NW
EW = RW * D
KI = RW // 128

_mesh = plsc.VectorSubcoreMesh(core_axis_name="c", subcore_axis_name="s")


@functools.partial(
    pl.kernel,
    out_type=jax.ShapeDtypeStruct((B * D,), jnp.float32),
    mesh=_mesh,
    scratch_types=[
        pltpu.VMEM((EW,), jnp.float32),
        pltpu.VMEM((EW,), jnp.float32),
        pltpu.VMEM((EW,), jnp.float32),
        pltpu.VMEM((RW,), jnp.int32),
        pltpu.VMEM((KI, 128), jnp.int32),
        pltpu.VMEM((RW,), jnp.float32),
        pltpu.VMEM((RW,), jnp.float32),
        pltpu.SemaphoreType.DMA,
        pltpu.SemaphoreType.DMA,
    ],
)
def _add_noise_sc(x_hbm, n_hbm, t_hbm, tab1_hbm, tab2_hbm, out_hbm,
                  x_v, n_v, o_v, t_v, idx_v, c1_v, c2_v, sem_in, sem_g):
    wid = lax.axis_index("s") * NC + lax.axis_index("c")
    row0 = wid * RW
    e0 = wid * EW

    cp_x = pltpu.async_copy(x_hbm.at[pl.ds(e0, EW)], x_v, sem_in)
    cp_n = pltpu.async_copy(n_hbm.at[pl.ds(e0, EW)], n_v, sem_in)

    pltpu.sync_copy(t_hbm.at[pl.ds(row0, RW)], t_v)

    for i in range(RW // L):
        tv = t_v[pl.ds(i * L, L)]
        rows = row0 + i * L + lax.iota(jnp.int32, L)
        k, c = divmod(i, 128 // L)
        idx_v[k, pl.ds(c * L, L)] = rows * T + tv

    gathers = []
    for k in range(KI):
        gathers.append(
            pltpu.async_copy(tab1_hbm.at[idx_v.at[k]],
                             c1_v.at[pl.ds(k * 128, 128)], sem_g))
        gathers.append(
            pltpu.async_copy(tab2_hbm.at[idx_v.at[k]],
                             c2_v.at[pl.ds(k * 128, 128)], sem_g))
    for g in gathers:
        g.wait()
    cp_x.wait()
    cp_n.wait()

    def group_body(g, carry):
        c1g = c1_v[pl.ds(g * L, L)]
        c2g = c2_v[pl.ds(g * L, L)]
        off0 = g * L * D
        for m in range(L):
            c1b = c1g[m]
            c2b = c2g[m]
            for q in range(D // L):
                off = off0 + m * D + q * L
                o_v[pl.ds(off, L)] = (c1b * x_v[pl.ds(off, L)]
                                      + c2b * n_v[pl.ds(off, L)])
        return carry

    lax.fori_loop(0, RW // L, group_body, 0)

    pltpu.sync_copy(o_v, out_hbm.at[pl.ds(e0, EW)])


def kernel(x_start, t, noise, sqrt_alphas_cumprod, sqrt_one_minus_alphas_cumprod):
    out = _add_noise_sc(
        x_start.reshape(B * D),
        noise.reshape(B * D),
        t.astype(jnp.int32),
        sqrt_alphas_cumprod.reshape(B * T),
        sqrt_one_minus_alphas_cumprod.reshape(B * T),
    )
    return out.reshape(B, D)


# speedup vs baseline: 9.5288x; 9.5288x over previous
import functools

import jax
import jax.numpy as jnp
from jax import lax
from jax.experimental import pallas as pl
from jax.experimental.pallas import tpu as pltpu
from jax.experimental.pallas import tpu_sc as plsc

B = 16384
D = 64
T = 1000

NC = 2
NS = 16
L = 16
NW = NC * NS
RW = B // NW
EW = RW * D
KI = RW // 128
DB = D // 8
CHUNK = KI * 1024

_mesh = plsc.VectorSubcoreMesh(core_axis_name="c", subcore_axis_name="s")


@functools.partial(
    pl.kernel,
    out_type=jax.ShapeDtypeStruct((B * D,), jnp.float32),
    mesh=_mesh,
    scratch_types=[
        pltpu.VMEM((EW,), jnp.float32),
        pltpu.VMEM((EW,), jnp.float32),
        pltpu.VMEM((EW,), jnp.float32),
        pltpu.VMEM((RW,), jnp.int32),
        pltpu.VMEM((KI, 128), jnp.int32),
        pltpu.VMEM((RW,), jnp.float32),
        pltpu.VMEM((RW,), jnp.float32),
        pltpu.SemaphoreType.DMA,
        pltpu.SemaphoreType.DMA,
        pltpu.SemaphoreType.DMA,
    ],
)
def _add_noise_sc(x_hbm, n_hbm, t_hbm, tab1_hbm, tab2_hbm, out_hbm,
                  x_v, n_v, o_v, t_v, idx_v, c1_v, c2_v,
                  sem_in, sem_g, sem_out):
    wid = lax.axis_index("s") * NC + lax.axis_index("c")
    row0 = wid * RW

    copies = []
    for db in range(DB):
        src = pl.ds(db * (B * 8) + wid * CHUNK, CHUNK)
        dst = pl.ds(db * CHUNK, CHUNK)
        copies.append(pltpu.async_copy(x_hbm.at[src], x_v.at[dst], sem_in))
        copies.append(pltpu.async_copy(n_hbm.at[src], n_v.at[dst], sem_in))

    pltpu.sync_copy(t_hbm.at[pl.ds(row0, RW)], t_v)

    for jc in range(RW // L):
        tv = t_v[pl.ds(jc * L, L)]
        lane = (jc % 8) * L + lax.iota(jnp.int32, L)
        band = wid * KI + jc // 8
        idx = ((tv >> 3) * (B * 8) + ((tv & 7) << 7)
               + (band * 1024 + lane))
        idx_v[jc // 8, pl.ds((jc % 8) * L, L)] = idx

    gathers = []
    for k in range(KI):
        gathers.append(
            pltpu.async_copy(tab1_hbm.at[idx_v.at[k]],
                             c1_v.at[pl.ds(k * 128, 128)], sem_g))
        gathers.append(
            pltpu.async_copy(tab2_hbm.at[idx_v.at[k]],
                             c2_v.at[pl.ds(k * 128, 128)], sem_g))
    for g in gathers:
        g.wait()
    for cp in copies:
        cp.wait()

    def band_body(db, carry):
        for ib in range(KI):
            c1r = [c1_v[pl.ds(ib * 128 + v * L, L)] for v in range(8)]
            c2r = [c2_v[pl.ds(ib * 128 + v * L, L)] for v in range(8)]
            for di in range(8):
                for v in range(8):
                    off = db * CHUNK + ib * 1024 + di * 128 + v * L
                    o_v[pl.ds(off, L)] = (c1r[v] * x_v[pl.ds(off, L)]
                                          + c2r[v] * n_v[pl.ds(off, L)])
        return carry

    lax.fori_loop(0, DB, band_body, 0)

    outs = []
    for db in range(DB):
        dst = pl.ds(db * (B * 8) + wid * CHUNK, CHUNK)
        src = pl.ds(db * CHUNK, CHUNK)
        outs.append(pltpu.async_copy(o_v.at[src], out_hbm.at[dst], sem_out))
    for cp in outs:
        cp.wait()


def _phys_flat(a, band):
    n, c = a.shape
    v = a.reshape(n // 128, 128, c // band, band)
    return v.transpose(2, 0, 3, 1).reshape(n * c)


def kernel(x_start, t, noise, sqrt_alphas_cumprod, sqrt_one_minus_alphas_cumprod):
    out = _add_noise_sc(
        _phys_flat(x_start, 8),
        _phys_flat(noise, 8),
        t.astype(jnp.int32),
        _phys_flat(sqrt_alphas_cumprod, 8),
        _phys_flat(sqrt_one_minus_alphas_cumprod, 8),
    )
    return (out.reshape(DB, B // 128, 8, 128)
               .transpose(1, 3, 0, 2)
               .reshape(B, D))
